# Initial kernel scaffold; baseline (speedup 1.0000x reference)
import jax
import jax.numpy as jnp
from jax.experimental import pallas as pl


def kernel(x, edge_index, batch, W0, b0, g0, be0, W1, b1, g1, be1, W2, b2, g2, be2, W3, b3, g3, be3, W4, b4, g4, be4, lw1, lb1, lw2, lb2):
    raise NotImplementedError("write your pallas kernel here")


# speedup vs baseline: 9.5136x; 9.5136x over previous
import functools

import jax
import jax.numpy as jnp
from jax import lax
from jax.experimental import pallas as pl
from jax.experimental.pallas import tpu as pltpu
from jax.experimental.pallas import tpu_sc as plsc

N = 10000
E = 320000
D = 128
G = 64
NL = 5

NC = 2
NS = 16
NW = NC * NS
EPW = E // NW
C = 80
NCHUNK = EPW // C
NPAD = 10240
RPS = NPAD // NS

_MESH = plsc.VectorSubcoreMesh(core_axis_name="c", subcore_axis_name="s",
                               num_cores=NC, num_subcores=NS)


def _deg_body(dst_hbm, zero_hbm, out_hbm, acc, idx_v, ones_v):
    cid = lax.axis_index("c")
    sid = lax.axis_index("s")
    wid = sid * NC + cid
    lane = lax.iota(jnp.int32, 16)
    one_row = jnp.where(lane == 0, 1.0, 0.0).astype(jnp.float32)

    def fill(i, carry):
        ones_v[i, :] = one_row
        return carry

    lax.fori_loop(0, C, fill, 0)
    pltpu.sync_copy(zero_hbm.at[pl.ds(sid * RPS, RPS)],
                    acc.at[pl.ds(sid * RPS, RPS)])
    plsc.subcore_barrier()

    def body(k, carry):
        base = wid * EPW + k * C
        pltpu.sync_copy(dst_hbm.at[pl.ds(base, C)], idx_v)
        pltpu.sync_copy(ones_v, acc.at[idx_v], add=True)
        return carry

    lax.fori_loop(0, NCHUNK, body, 0)
    plsc.subcore_barrier()
    pltpu.sync_copy(acc.at[pl.ds(sid * RPS, RPS)],
                    out_hbm.at[cid, pl.ds(sid * RPS, RPS)])


_deg_call = functools.partial(
    pl.kernel,
    out_type=jax.ShapeDtypeStruct((NC, NPAD, 16), jnp.float32),
    mesh=_MESH,
    scratch_types=[
        pltpu.VMEM_SHARED((NPAD, 16), jnp.float32),
        pltpu.VMEM((C,), jnp.int32),
        pltpu.VMEM((C, 16), jnp.float32),
    ],
)(_deg_body)


def _scat_body(g_hbm, src_hbm, dst_hbm, zero_hbm, out_hbm,
               acc, sidx, didx, rows, sem):
    cid = lax.axis_index("c")
    sid = lax.axis_index("s")
    wid = sid * NC + cid
    pltpu.sync_copy(zero_hbm.at[pl.ds(sid * RPS, RPS)],
                    acc.at[pl.ds(sid * RPS, RPS)])
    plsc.subcore_barrier()

    def body(k, carry):
        base = wid * EPW + k * C
        pltpu.sync_copy(src_hbm.at[pl.ds(base, C)], sidx)
        pltpu.sync_copy(dst_hbm.at[pl.ds(base, C)], didx)
        pltpu.async_copy(g_hbm.at[sidx], rows, sem).wait()
        pltpu.sync_copy(rows, acc.at[didx], add=True)
        return carry

    lax.fori_loop(0, NCHUNK, body, 0)
    plsc.subcore_barrier()
    pltpu.sync_copy(acc.at[pl.ds(sid * RPS, RPS)],
                    out_hbm.at[cid, pl.ds(sid * RPS, RPS)])


_scat_call = functools.partial(
    pl.kernel,
    out_type=jax.ShapeDtypeStruct((NC, NPAD, D), jnp.float32),
    mesh=_MESH,
    scratch_types=[
        pltpu.VMEM_SHARED((NPAD, D), jnp.float32),
        pltpu.VMEM((C,), jnp.int32),
        pltpu.VMEM((C,), jnp.int32),
        pltpu.VMEM((C, D), jnp.float32),
        pltpu.SemaphoreType.DMA,
    ],
)(_scat_body)


def _dinv_from(dp_ref):
    deg = dp_ref[0, :N, 0:1] + dp_ref[1, :N, 0:1] + 1.0
    return lax.rsqrt(deg)


def _first_body(x_ref, w_ref, dp_ref, g_ref):
    dinv = _dinv_from(dp_ref)
    h = jnp.dot(x_ref[...], w_ref[...], preferred_element_type=jnp.float32)
    g_ref[...] = h * dinv


def _tc_first(x, W0, degp):
    return pl.pallas_call(
        _first_body,
        out_shape=jax.ShapeDtypeStruct((N, D), jnp.float32),
    )(x, W0, degp)


def _layer_post(sp_ref, g_ref, xr_ref, dp_ref, b_ref, gm_ref, be_ref, use_res):
    dinv = _dinv_from(dp_ref)
    agg = (sp_ref[0, :N] + sp_ref[1, :N] + g_ref[...]) * dinv + b_ref[...]
    mu = jnp.mean(agg, axis=0, keepdims=True)
    var = jnp.mean((agg - mu) ** 2, axis=0, keepdims=True)
    xn = (agg - mu) * lax.rsqrt(var + 1e-5) * gm_ref[...] + be_ref[...]
    xn = jnp.maximum(xn, 0.0)
    if use_res:
        xn = xn + xr_ref[...]
    return xn, dinv


def _mid_body(use_res, sp_ref, g_ref, xr_ref, dp_ref, b_ref, gm_ref, be_ref,
              w_ref, xo_ref, go_ref):
    xn, dinv = _layer_post(sp_ref, g_ref, xr_ref, dp_ref, b_ref, gm_ref,
                           be_ref, use_res)
    xo_ref[...] = xn
    go_ref[...] = jnp.dot(xn, w_ref[...],
                          preferred_element_type=jnp.float32) * dinv


def _tc_mid(scatp, g, xres, degp, b, gm, be, Wn, use_res):
    return pl.pallas_call(
        functools.partial(_mid_body, use_res),
        out_shape=(jax.ShapeDtypeStruct((N, D), jnp.float32),
                   jax.ShapeDtypeStruct((N, D), jnp.float32)),
    )(scatp, g, xres, degp, b, gm, be, Wn)


def _last_body(sp_ref, g_ref, xr_ref, dp_ref, b_ref, gm_ref, be_ref, bt_ref,
               lw1_ref, lb1_ref, lw2_ref, lb2_ref, o_ref):
    xn, _ = _layer_post(sp_ref, g_ref, xr_ref, dp_ref, b_ref, gm_ref,
                        be_ref, True)
    gid = lax.broadcasted_iota(jnp.int32, (G, N), 0)
    onehot = (bt_ref[...] == gid).astype(jnp.float32)
    sums = jnp.dot(onehot, xn, preferred_element_type=jnp.float32)
    cnt = jnp.sum(onehot, axis=1, keepdims=True)
    pooled = sums / jnp.maximum(cnt, 1.0)
    h1 = jnp.dot(pooled, lw1_ref[...], preferred_element_type=jnp.float32)
    h1 = jnp.maximum(h1 + lb1_ref[...], 0.0)
    o_ref[...] = jnp.dot(h1, lw2_ref[...],
                         preferred_element_type=jnp.float32) + lb2_ref[...]


def _tc_last(scatp, g, xres, degp, b, gm, be, batch2, lw1, lb1, lw2, lb2):
    return pl.pallas_call(
        _last_body,
        out_shape=jax.ShapeDtypeStruct((G, 1), jnp.float32),
    )(scatp, g, xres, degp, b, gm, be, batch2, lw1, lb1, lw2, lb2)


def kernel(x, edge_index, batch, W0, b0, g0, be0, W1, b1, g1, be1,
           W2, b2, g2, be2, W3, b3, g3, be3, W4, b4, g4, be4,
           lw1, lb1, lw2, lb2):
    Ws = [W0, W1, W2, W3, W4]
    bs = [b0.reshape(1, D), b1.reshape(1, D), b2.reshape(1, D),
          b3.reshape(1, D), b4.reshape(1, D)]
    gms = [g0.reshape(1, D), g1.reshape(1, D), g2.reshape(1, D),
           g3.reshape(1, D), g4.reshape(1, D)]
    bes = [be0.reshape(1, D), be1.reshape(1, D), be2.reshape(1, D),
           be3.reshape(1, D), be4.reshape(1, D)]
    src = edge_index[0]
    dst = edge_index[1]
    zeros_nd = jnp.zeros((NPAD, D), jnp.float32)
    zeros_n16 = jnp.zeros((NPAD, 16), jnp.float32)
    batch2 = batch.reshape(1, N)

    degp = _deg_call(dst, zeros_n16)
    g = _tc_first(x, Ws[0], degp)
    xcur = x
    for i in range(NL):
        scatp = _scat_call(g, src, dst, zeros_nd)
        if i < NL - 1:
            xcur, g = _tc_mid(scatp, g, xcur, degp, bs[i], gms[i], bes[i],
                              Ws[i + 1], use_res=(i > 0))
        else:
            out = _tc_last(scatp, g, xcur, degp, bs[i], gms[i], bes[i],
                           batch2, lw1, lb1, lw2.reshape(D // 2, 1),
                           lb2.reshape(1, 1))
    return out


# speedup vs baseline: 16.1284x; 1.6953x over previous
import functools

import jax
import jax.numpy as jnp
from jax import lax
from jax.experimental import pallas as pl
from jax.experimental.pallas import tpu as pltpu
from jax.experimental.pallas import tpu_sc as plsc

N = 10000
E = 320000
D = 128
G = 64
NL = 5

NC = 2
NS = 16
NW = NC * NS
C = 80
EPW = E // NW
NCHUNK = EPW // C
NPAD = 10240
RPS = NPAD // NS

_MESH = plsc.VectorSubcoreMesh(core_axis_name="c", subcore_axis_name="s",
                               num_cores=NC, num_subcores=NS)


def _scat_body(g_hbm, src_hbm, dst_hbm, zero_hbm, out_hbm,
               acc, sidx0, sidx1, didx0, didx1, rows0, rows1,
               semI0, semI1, semG0, semG1):
    cid = lax.axis_index("c")
    sid = lax.axis_index("s")
    wid = sid * NC + cid
    base0 = wid * EPW
    pltpu.sync_copy(zero_hbm.at[pl.ds(sid * RPS, RPS)],
                    acc.at[pl.ds(sid * RPS, RPS)])
    plsc.subcore_barrier()

    pltpu.sync_copy(src_hbm.at[pl.ds(base0, C)], sidx0)
    pltpu.sync_copy(dst_hbm.at[pl.ds(base0, C)], didx0)
    pltpu.async_copy(g_hbm.at[sidx0], rows0, semG0)
    pltpu.async_copy(src_hbm.at[pl.ds(base0 + C, C)], sidx1, semI1)
    pltpu.async_copy(dst_hbm.at[pl.ds(base0 + C, C)], didx1, semI1)

    def body(j, carry):
        k0 = 2 * j
        b1 = base0 + (k0 + 1) * C
        b2 = base0 + (k0 + 2) * C
        pltpu.make_async_copy(src_hbm.at[pl.ds(b1, C)], sidx1, semI1).wait()
        pltpu.make_async_copy(dst_hbm.at[pl.ds(b1, C)], didx1, semI1).wait()
        pltpu.async_copy(g_hbm.at[sidx1], rows1, semG1)
        pltpu.make_async_copy(g_hbm.at[sidx0], rows0, semG0).wait()
        pltpu.sync_copy(rows0, acc.at[didx0], add=True)
        pltpu.async_copy(src_hbm.at[pl.ds(b2, C)], sidx0, semI0)
        pltpu.async_copy(dst_hbm.at[pl.ds(b2, C)], didx0, semI0)
        pltpu.make_async_copy(src_hbm.at[pl.ds(b2, C)], sidx0, semI0).wait()
        pltpu.make_async_copy(dst_hbm.at[pl.ds(b2, C)], didx0, semI0).wait()
        pltpu.async_copy(g_hbm.at[sidx0], rows0, semG0)
        pltpu.make_async_copy(g_hbm.at[sidx1], rows1, semG1).wait()
        pltpu.sync_copy(rows1, acc.at[didx1], add=True)

        @pl.when(k0 + 3 < NCHUNK)
        def _():
            b3 = base0 + (k0 + 3) * C
            pltpu.async_copy(src_hbm.at[pl.ds(b3, C)], sidx1, semI1)
            pltpu.async_copy(dst_hbm.at[pl.ds(b3, C)], didx1, semI1)

        return carry

    lax.fori_loop(0, NCHUNK // 2, body, 0)
    pltpu.make_async_copy(g_hbm.at[sidx0], rows0, semG0).wait()
    pltpu.sync_copy(rows0, acc.at[didx0], add=True)
    plsc.subcore_barrier()
    pltpu.sync_copy(acc.at[pl.ds(sid * RPS, RPS)],
                    out_hbm.at[cid, pl.ds(sid * RPS, RPS)])


_scat_call = functools.partial(
    pl.kernel,
    out_type=jax.ShapeDtypeStruct((NC, NPAD, D), jnp.float32),
    mesh=_MESH,
    scratch_types=[
        pltpu.VMEM_SHARED((NPAD, D), jnp.float32),
        pltpu.VMEM((C,), jnp.int32),
        pltpu.VMEM((C,), jnp.int32),
        pltpu.VMEM((C,), jnp.int32),
        pltpu.VMEM((C,), jnp.int32),
        pltpu.VMEM((C, D), jnp.float32),
        pltpu.VMEM((C, D), jnp.float32),
        pltpu.SemaphoreType.DMA,
        pltpu.SemaphoreType.DMA,
        pltpu.SemaphoreType.DMA,
        pltpu.SemaphoreType.DMA,
    ],
)(_scat_body)


BM = 2000


def _dinv_body(dp_ref, o_ref):
    deg = dp_ref[0, :N, 0:1] + dp_ref[1, :N, 0:1] + 1.0
    o_ref[...] = jnp.broadcast_to(lax.rsqrt(deg), (N, D))


def _dinv_call(degp):
    return pl.pallas_call(
        _dinv_body,
        out_shape=jax.ShapeDtypeStruct((N, D), jnp.float32),
    )(degp)


def _mm_body(x_ref, w_ref, dv_ref, o_ref):
    o_ref[...] = jnp.dot(x_ref[...], w_ref[...],
                         preferred_element_type=jnp.float32,
                         precision=jax.lax.Precision.HIGHEST) * dv_ref[...]


def _matmul_g(x, W, dinv_b):
    return pl.pallas_call(
        _mm_body,
        grid=(N // BM,),
        in_specs=[pl.BlockSpec((BM, D), lambda i: (i, 0)),
                  pl.BlockSpec((D, D), lambda i: (0, 0)),
                  pl.BlockSpec((BM, D), lambda i: (i, 0))],
        out_specs=pl.BlockSpec((BM, D), lambda i: (i, 0)),
        out_shape=jax.ShapeDtypeStruct((N, D), jnp.float32),
    )(x, W, dinv_b)


def _layer_post(sp_ref, g_ref, xr_ref, dv_ref, b_ref, gm_ref, be_ref, use_res):
    dinv = dv_ref[...]
    agg = (sp_ref[0, :N] + sp_ref[1, :N] + g_ref[...]) * dinv + b_ref[...]
    mu = jnp.mean(agg, axis=0, keepdims=True)
    var = jnp.mean((agg - mu) ** 2, axis=0, keepdims=True)
    xn = (agg - mu) * lax.rsqrt(var + 1e-5) * gm_ref[...] + be_ref[...]
    xn = jnp.maximum(xn, 0.0)
    if use_res:
        xn = xn + xr_ref[...]
    return xn


def _bn_body(use_res, sp_ref, g_ref, xr_ref, dv_ref, b_ref, gm_ref, be_ref,
             xo_ref):
    xo_ref[...] = _layer_post(sp_ref, g_ref, xr_ref, dv_ref, b_ref, gm_ref,
                              be_ref, use_res)


def _tc_bn(scatp, g, xres, dinv_b, b, gm, be, use_res):
    return pl.pallas_call(
        functools.partial(_bn_body, use_res),
        out_shape=jax.ShapeDtypeStruct((N, D), jnp.float32),
    )(scatp, g, xres, dinv_b, b, gm, be)


def _last_body(sp_ref, g_ref, xr_ref, dv_ref, b_ref, gm_ref, be_ref, bt_ref,
               lw1_ref, lb1_ref, lw2_ref, lb2_ref, o_ref):
    xn = _layer_post(sp_ref, g_ref, xr_ref, dv_ref, b_ref, gm_ref,
                     be_ref, True)
    gid = lax.broadcasted_iota(jnp.int32, (G, N), 0)
    onehot = (bt_ref[...] == gid).astype(jnp.float32)
    sums = jnp.dot(onehot, xn, preferred_element_type=jnp.float32)
    cnt = jnp.sum(onehot, axis=1, keepdims=True)
    pooled = sums / jnp.maximum(cnt, 1.0)
    h1 = jnp.dot(pooled, lw1_ref[...], preferred_element_type=jnp.float32)
    h1 = jnp.maximum(h1 + lb1_ref[...], 0.0)
    o_ref[...] = jnp.dot(h1, lw2_ref[...],
                         preferred_element_type=jnp.float32) + lb2_ref[...]


def _tc_last(scatp, g, xres, dinv_b, b, gm, be, batch2, lw1, lb1, lw2, lb2):
    return pl.pallas_call(
        _last_body,
        out_shape=jax.ShapeDtypeStruct((G, 1), jnp.float32),
    )(scatp, g, xres, dinv_b, b, gm, be, batch2, lw1, lb1, lw2, lb2)


def kernel(x, edge_index, batch, W0, b0, g0, be0, W1, b1, g1, be1,
           W2, b2, g2, be2, W3, b3, g3, be3, W4, b4, g4, be4,
           lw1, lb1, lw2, lb2):
    Ws = [W0, W1, W2, W3, W4]
    bs = [b0.reshape(1, D), b1.reshape(1, D), b2.reshape(1, D),
          b3.reshape(1, D), b4.reshape(1, D)]
    gms = [g0.reshape(1, D), g1.reshape(1, D), g2.reshape(1, D),
           g3.reshape(1, D), g4.reshape(1, D)]
    bes = [be0.reshape(1, D), be1.reshape(1, D), be2.reshape(1, D),
           be3.reshape(1, D), be4.reshape(1, D)]
    src = edge_index[0]
    dst = edge_index[1]
    zeros_nd = jnp.zeros((NPAD, D), jnp.float32)
    batch2 = batch.reshape(1, N)

    ones_nd = jnp.ones((N, D), jnp.float32)
    degp = _scat_call(ones_nd, src, dst, zeros_nd)
    dinv_b = _dinv_call(degp)
    g = _matmul_g(x, Ws[0], dinv_b)
    xcur = x
    for i in range(NL):
        scatp = _scat_call(g, src, dst, zeros_nd)
        if i < NL - 1:
            xcur = _tc_bn(scatp, g, xcur, dinv_b, bs[i], gms[i], bes[i],
                          use_res=(i > 0))
            g = _matmul_g(xcur, Ws[i + 1], dinv_b)
        else:
            out = _tc_last(scatp, g, xcur, dinv_b, bs[i], gms[i], bes[i],
                           batch2, lw1, lb1, lw2.reshape(D // 2, 1),
                           lb2.reshape(1, 1))
    return out
